# baseline (speedup 1.0000x reference)
import jax
import jax.numpy as jnp
from jax.experimental import pallas as pl
from jax.experimental.pallas import tpu as pltpu

N = 4096
D = 256
K = 16
INV_T = 10.0
BLOCK = 128
NEG = -3.0
BIG = 4.0


def _norm_body(x_ref, o_ref):
    x = x_ref[...]
    n = jnp.maximum(jnp.sqrt(jnp.sum(x * x, axis=-1, keepdims=True)), 1e-12)
    o_ref[...] = x / n


def _body(xb_ref, xf_ref, o_ref):
    xb = xb_ref[...]
    xf = xf_ref[...]
    sim = jax.lax.dot_general(
        xb, xf, (((1,), (1,)), ((), ())), preferred_element_type=jnp.float32
    )

    r3 = sim.reshape(BLOCK, N // 128, 128)
    t1 = jnp.max(r3, axis=1)
    w3 = jnp.where(r3 == t1[:, None, :], NEG, r3)
    t2 = jnp.max(w3, axis=1)
    w3 = jnp.where(w3 == t2[:, None, :], NEG, w3)
    t3 = jnp.max(w3, axis=1)
    w3 = jnp.where(w3 == t3[:, None, :], NEG, w3)
    t4 = jnp.max(w3, axis=1)
    cand = jnp.concatenate([t1, t2, t3, t4], axis=-1)

    m0 = jnp.max(t1, axis=-1, keepdims=True)
    w = cand
    t = m0
    for _ in range(K - 1):
        w = jnp.where(w >= t, NEG, w)
        t = jnp.max(w, axis=-1, keepdims=True)

    count = jnp.sum(jnp.where(sim >= t, 1.0, 0.0), axis=-1, keepdims=True)

    def _cond(state):
        _, count_, it = state
        return jnp.logical_and(jnp.any(count_ > float(K)), it < 24)

    def _repair(state):
        t_, count_, it = state
        bad = count_ > float(K)
        m = jnp.min(jnp.where(sim >= t_, sim, BIG), axis=-1, keepdims=True)
        tn = jnp.min(jnp.where(sim > m, sim, BIG), axis=-1, keepdims=True)
        t2_ = jnp.where(jnp.logical_and(bad, tn < BIG), tn, t_)
        c2 = jnp.sum(jnp.where(sim >= t2_, 1.0, 0.0), axis=-1, keepdims=True)
        return t2_, c2, it + 1

    t, count, _ = jax.lax.while_loop(_cond, _repair, (t, count, 0))

    e = jnp.where(sim >= t, jnp.exp((sim - m0) * INV_T), 0.0)
    s = jnp.sum(e, axis=-1, keepdims=True)
    o_ref[...] = e * (1.0 / s)


def kernel(X_c):
    Xn = pl.pallas_call(
        _norm_body,
        grid=(4,),
        in_specs=[pl.BlockSpec((N // 4, D), lambda i: (i, 0))],
        out_specs=pl.BlockSpec((N // 4, D), lambda i: (i, 0)),
        out_shape=jax.ShapeDtypeStruct((N, D), jnp.float32),
    )(X_c)
    return pl.pallas_call(
        _body,
        grid=(N // BLOCK,),
        in_specs=[
            pl.BlockSpec((BLOCK, D), lambda i: (i, 0)),
            pl.BlockSpec((N, D), lambda i: (0, 0)),
        ],
        out_specs=pl.BlockSpec((BLOCK, N), lambda i: (i, 0)),
        out_shape=jax.ShapeDtypeStruct((N, N), jnp.float32),
        compiler_params=pltpu.CompilerParams(
            dimension_semantics=("arbitrary",),
        ),
    )(Xn, Xn)


# speedup vs baseline: 1.2485x; 1.2485x over previous
import jax
import jax.numpy as jnp
from jax.experimental import pallas as pl
from jax.experimental.pallas import tpu as pltpu

N = 4096
D = 256
K = 16
INV_T = 10.0
BLOCK = 256
NEG = -3.0
BIG = 4.0


def _norm_body(x_ref, o_ref):
    x = x_ref[...]
    n = jnp.maximum(jnp.sqrt(jnp.sum(x * x, axis=-1, keepdims=True)), 1e-12)
    o_ref[...] = x / n


def _body(xb_ref, xf_ref, o_ref):
    xb = xb_ref[...]
    xf = xf_ref[...]
    sim = jax.lax.dot_general(
        xb, xf, (((1,), (1,)), ((), ())), preferred_element_type=jnp.float32
    )

    r3 = sim.reshape(BLOCK, N // 128, 128)
    t1 = jnp.max(r3, axis=1)
    w3 = jnp.where(r3 == t1[:, None, :], NEG, r3)
    t2 = jnp.max(w3, axis=1)
    w3 = jnp.where(w3 == t2[:, None, :], NEG, w3)
    t3 = jnp.max(w3, axis=1)
    w3 = jnp.where(w3 == t3[:, None, :], NEG, w3)
    t4 = jnp.max(w3, axis=1)
    cand = jnp.concatenate([t1, t2, t3, t4], axis=-1)

    m0 = jnp.max(t1, axis=-1, keepdims=True)
    w = cand
    t = m0
    for _ in range(K - 1):
        w = jnp.where(w >= t, NEG, w)
        t = jnp.max(w, axis=-1, keepdims=True)

    count = jnp.sum(jnp.where(sim >= t, 1.0, 0.0), axis=-1, keepdims=True)

    def _cond(state):
        _, count_, it = state
        return jnp.logical_and(jnp.any(count_ > float(K)), it < 24)

    def _repair(state):
        t_, count_, it = state
        bad = count_ > float(K)
        m = jnp.min(jnp.where(sim >= t_, sim, BIG), axis=-1, keepdims=True)
        tn = jnp.min(jnp.where(sim > m, sim, BIG), axis=-1, keepdims=True)
        t2_ = jnp.where(jnp.logical_and(bad, tn < BIG), tn, t_)
        c2 = jnp.sum(jnp.where(sim >= t2_, 1.0, 0.0), axis=-1, keepdims=True)
        return t2_, c2, it + 1

    t, count, _ = jax.lax.while_loop(_cond, _repair, (t, count, 0))

    e = jnp.where(sim >= t, jnp.exp((sim - m0) * INV_T), 0.0)
    s = jnp.sum(e, axis=-1, keepdims=True)
    o_ref[...] = e * (1.0 / s)


def kernel(X_c):
    Xn = pl.pallas_call(
        _norm_body,
        grid=(4,),
        in_specs=[pl.BlockSpec((N // 4, D), lambda i: (i, 0))],
        out_specs=pl.BlockSpec((N // 4, D), lambda i: (i, 0)),
        out_shape=jax.ShapeDtypeStruct((N, D), jnp.float32),
    )(X_c)
    return pl.pallas_call(
        _body,
        grid=(N // BLOCK,),
        in_specs=[
            pl.BlockSpec((BLOCK, D), lambda i: (i, 0)),
            pl.BlockSpec((N, D), lambda i: (0, 0)),
        ],
        out_specs=pl.BlockSpec((BLOCK, N), lambda i: (i, 0)),
        out_shape=jax.ShapeDtypeStruct((N, N), jnp.float32),
        compiler_params=pltpu.CompilerParams(
            dimension_semantics=("arbitrary",),
        ),
    )(Xn, Xn)


# speedup vs baseline: 1.3520x; 1.0829x over previous
import jax
import jax.numpy as jnp
from jax.experimental import pallas as pl
from jax.experimental.pallas import tpu as pltpu

N = 4096
D = 256
K = 16
INV_T = 10.0
BLOCK = 512
NEG = -3.0
BIG = 4.0


def _norm_body(x_ref, o_ref):
    x = x_ref[...]
    n = jnp.maximum(jnp.sqrt(jnp.sum(x * x, axis=-1, keepdims=True)), 1e-12)
    o_ref[...] = x / n


def _body(xb_ref, xf_ref, o_ref):
    xb = xb_ref[...]
    xf = xf_ref[...]
    sim = jax.lax.dot_general(
        xb, xf, (((1,), (1,)), ((), ())), preferred_element_type=jnp.float32
    )

    r3 = sim.reshape(BLOCK, N // 128, 128)
    t1 = jnp.max(r3, axis=1)
    w3 = jnp.where(r3 == t1[:, None, :], NEG, r3)
    t2 = jnp.max(w3, axis=1)
    w3 = jnp.where(w3 == t2[:, None, :], NEG, w3)
    t3 = jnp.max(w3, axis=1)
    w3 = jnp.where(w3 == t3[:, None, :], NEG, w3)
    t4 = jnp.max(w3, axis=1)
    cand = jnp.concatenate([t1, t2, t3, t4], axis=-1)

    m0 = jnp.max(t1, axis=-1, keepdims=True)
    w = cand
    t = m0
    for _ in range(K - 1):
        w = jnp.where(w >= t, NEG, w)
        t = jnp.max(w, axis=-1, keepdims=True)

    count = jnp.sum(jnp.where(sim >= t, 1.0, 0.0), axis=-1, keepdims=True)

    def _cond(state):
        _, count_, it = state
        return jnp.logical_and(jnp.any(count_ > float(K)), it < 24)

    def _repair(state):
        t_, count_, it = state
        bad = count_ > float(K)
        m = jnp.min(jnp.where(sim >= t_, sim, BIG), axis=-1, keepdims=True)
        tn = jnp.min(jnp.where(sim > m, sim, BIG), axis=-1, keepdims=True)
        t2_ = jnp.where(jnp.logical_and(bad, tn < BIG), tn, t_)
        c2 = jnp.sum(jnp.where(sim >= t2_, 1.0, 0.0), axis=-1, keepdims=True)
        return t2_, c2, it + 1

    t, count, _ = jax.lax.while_loop(_cond, _repair, (t, count, 0))

    e = jnp.where(sim >= t, jnp.exp((sim - m0) * INV_T), 0.0)
    s = jnp.sum(e, axis=-1, keepdims=True)
    o_ref[...] = e * (1.0 / s)


def kernel(X_c):
    Xn = pl.pallas_call(
        _norm_body,
        grid=(4,),
        in_specs=[pl.BlockSpec((N // 4, D), lambda i: (i, 0))],
        out_specs=pl.BlockSpec((N // 4, D), lambda i: (i, 0)),
        out_shape=jax.ShapeDtypeStruct((N, D), jnp.float32),
    )(X_c)
    return pl.pallas_call(
        _body,
        grid=(N // BLOCK,),
        in_specs=[
            pl.BlockSpec((BLOCK, D), lambda i: (i, 0)),
            pl.BlockSpec((N, D), lambda i: (0, 0)),
        ],
        out_specs=pl.BlockSpec((BLOCK, N), lambda i: (i, 0)),
        out_shape=jax.ShapeDtypeStruct((N, N), jnp.float32),
        compiler_params=pltpu.CompilerParams(
            dimension_semantics=("arbitrary",),
        ),
    )(Xn, Xn)


# speedup vs baseline: 1.7434x; 1.2894x over previous
import functools

import jax
import jax.numpy as jnp
from jax.experimental import pallas as pl
from jax.experimental.pallas import tpu as pltpu

N = 4096
D = 256
K = 16
INV_T = 10.0
BLOCK = 512
NEG = -3.0
BIG = 4.0
LOG2E = 1.4426950408889634
S = N // 128


def _norm_body(x_ref, o_ref):
    x = x_ref[...]
    n = jnp.maximum(jnp.sqrt(jnp.sum(x * x, axis=-1, keepdims=True)), 1e-12)
    o_ref[...] = x / n


def _sl(x, g):
    return x[:, g * 128:(g + 1) * 128]


def _body(xb_ref, xf_ref, o_ref):
    xb = xb_ref[...]
    xf = xf_ref[...]
    sim = jax.lax.dot_general(
        xb, xf, (((1,), (1,)), ((), ())), preferred_element_type=jnp.float32
    )

    ws = [_sl(sim, g) for g in range(S)]
    t1 = functools.reduce(jnp.maximum, ws)
    cands = [t1]
    tp = t1
    for _lvl in range(3):
        ws = [jnp.where(wg == tp, NEG, wg) for wg in ws]
        tp = functools.reduce(jnp.maximum, ws)
        cands.append(tp)
    cand = jnp.concatenate(cands, axis=-1)

    m0 = jnp.max(t1, axis=-1, keepdims=True)
    w = cand
    t = m0
    for _ in range(K - 1):
        w = jnp.where(w >= t, NEG, w)
        t = jnp.max(w, axis=-1, keepdims=True)

    cacc = functools.reduce(
        jnp.add, [jnp.where(_sl(sim, g) >= t, 1.0, 0.0) for g in range(S)]
    )
    count = jnp.sum(cacc, axis=-1, keepdims=True)
    anybad0 = jnp.any(count > float(K))

    def _cond(state):
        _, count_, it = state
        return jnp.logical_and(jnp.any(count_ > float(K)), it < 24)

    def _repair(state):
        t_, count_, it = state
        bad = count_ > float(K)
        macc = functools.reduce(
            jnp.minimum,
            [jnp.where(_sl(sim, g) >= t_, _sl(sim, g), BIG) for g in range(S)],
        )
        m = jnp.min(macc, axis=-1, keepdims=True)
        nacc = functools.reduce(
            jnp.minimum,
            [jnp.where(_sl(sim, g) > m, _sl(sim, g), BIG) for g in range(S)],
        )
        tn = jnp.min(nacc, axis=-1, keepdims=True)
        t2_ = jnp.where(jnp.logical_and(bad, tn < BIG), tn, t_)
        c2acc = functools.reduce(
            jnp.add, [jnp.where(_sl(sim, g) >= t2_, 1.0, 0.0) for g in range(S)]
        )
        c2 = jnp.sum(c2acc, axis=-1, keepdims=True)
        return t2_, c2, it + 1

    t, count, _ = jax.lax.while_loop(_cond, _repair, (t, count, 0))

    e_cand = jnp.where(cand >= t, jnp.exp((cand - m0) * INV_T), 0.0)
    s = jnp.sum(e_cand, axis=-1, keepdims=True)

    def _s_full():
        eacc = functools.reduce(
            jnp.add,
            [
                jnp.where(
                    _sl(sim, g) >= t, jnp.exp((_sl(sim, g) - m0) * INV_T), 0.0
                )
                for g in range(S)
            ],
        )
        return jnp.sum(eacc, axis=-1, keepdims=True)

    s = jax.lax.cond(anybad0, _s_full, lambda: s)

    c1 = INV_T * LOG2E
    bias = m0 * c1 + jnp.log2(s)
    for g in range(S):
        sg = _sl(sim, g)
        o_ref[:, g * 128:(g + 1) * 128] = jnp.where(
            sg >= t, jnp.exp2(sg * c1 - bias), 0.0
        )


def kernel(X_c):
    Xn = pl.pallas_call(
        _norm_body,
        grid=(4,),
        in_specs=[pl.BlockSpec((N // 4, D), lambda i: (i, 0))],
        out_specs=pl.BlockSpec((N // 4, D), lambda i: (i, 0)),
        out_shape=jax.ShapeDtypeStruct((N, D), jnp.float32),
    )(X_c)
    return pl.pallas_call(
        _body,
        grid=(N // BLOCK,),
        in_specs=[
            pl.BlockSpec((BLOCK, D), lambda i: (i, 0)),
            pl.BlockSpec((N, D), lambda i: (0, 0)),
        ],
        out_specs=pl.BlockSpec((BLOCK, N), lambda i: (i, 0)),
        out_shape=jax.ShapeDtypeStruct((N, N), jnp.float32),
        compiler_params=pltpu.CompilerParams(
            dimension_semantics=("arbitrary",),
        ),
    )(Xn, Xn)
